# baseline (speedup 1.0000x reference)
import functools

import jax
import jax.numpy as jnp
from jax import lax
from jax.experimental import pallas as pl
from jax.experimental.pallas import tpu as pltpu
from jax.experimental.pallas import tpu_sc as plsc

_NC = 2
_NS = 16
_NW = _NC * _NS
_LANES = 16

_R = 8
_NBUF = 8


def kernel(x, W):
    b, l, d = x.shape
    assert 2 * b == _NBUF
    l_per_w = l // _NW
    n_chunks = l_per_w // _R
    units = n_chunks * b
    jpr = d // _LANES

    mesh = plsc.VectorSubcoreMesh(core_axis_name="c", subcore_axis_name="s")

    @functools.partial(
        pl.kernel,
        mesh=mesh,
        out_type=jax.ShapeDtypeStruct((b, l, d), jnp.float32),
        scratch_types=[
            pltpu.VMEM((_NBUF, _R, d), jnp.float32),
            pltpu.VMEM((2, _R, d), jnp.float32),
        ] + [pltpu.SemaphoreType.DMA] * (2 * _NBUF + 2),
    )
    def sc_add(x_hbm, w_hbm, o_hbm, x_ring, w_ring, *sems):
        in_sems = sems[:_NBUF]
        out_sems = sems[_NBUF:2 * _NBUF]
        w_sems = sems[2 * _NBUF:]

        wid = lax.axis_index("s") * _NC + lax.axis_index("c")
        l_base = wid * l_per_w

        def start_in(u, slot):
            c = u // b
            bb = u % b
            lo = l_base + c * _R
            pltpu.async_copy(x_hbm.at[bb, pl.ds(lo, _R)], x_ring.at[slot],
                             in_sems[slot])

        def wait_in(slot):
            pltpu.make_async_copy(x_hbm.at[0, pl.ds(0, _R)],
                                  x_ring.at[slot], in_sems[slot]).wait()

        def start_out(u, slot):
            c = u // b
            bb = u % b
            lo = l_base + c * _R
            pltpu.async_copy(x_ring.at[slot], o_hbm.at[bb, pl.ds(lo, _R)],
                             out_sems[slot])

        def wait_out(slot):
            pltpu.make_async_copy(x_ring.at[slot],
                                  o_hbm.at[0, pl.ds(0, _R)],
                                  out_sems[slot]).wait()

        def start_w(c, slot):
            pltpu.async_copy(w_hbm.at[pl.ds(l_base + c * _R, _R)],
                             w_ring.at[slot], w_sems[slot])

        def wait_w(slot):
            pltpu.make_async_copy(w_hbm.at[pl.ds(0, _R)],
                                  w_ring.at[slot], w_sems[slot]).wait()

        start_w(0, 0)
        start_w(1, 1)
        for u0 in range(_NBUF - 1):
            start_in(u0, u0)

        @pl.loop(0, n_chunks, step=2)
        def block_body(c_base):
            for j in range(2 * b):
                u = c_base * b + j
                c = c_base + j // b
                bb = j % b
                wslot = (j // b) % 2
                xslot = j % _NBUF
                if bb == 0:
                    wait_w(wslot)

                wait_in(xslot)

                @plsc.parallel_loop(0, _R, 1, unroll=2)
                def add_body(r):
                    for jj in range(jpr):
                        sl = pl.ds(jj * _LANES, _LANES)
                        plsc.addupdate(x_ring.at[xslot, r, sl],
                                       w_ring[wslot, r, sl])

                if bb == b - 1:
                    @pl.when(c + 2 < n_chunks)
                    def _():
                        start_w(c + 2, wslot)

                start_out(u, xslot)

                v = u + _NBUF - 1
                vslot = (j + _NBUF - 1) % _NBUF

                @pl.when(v < units)
                def _():
                    @pl.when(u > 0)
                    def _():
                        wait_out(vslot)

                    start_in(v, vslot)

        for u in range(units - _NBUF, units):
            wait_out(u % _NBUF)

    return sc_add(x, W)


# speedup vs baseline: 1.5697x; 1.5697x over previous
import functools

import jax
import jax.numpy as jnp
from jax import lax
from jax.experimental import pallas as pl
from jax.experimental.pallas import tpu as pltpu
from jax.experimental.pallas import tpu_sc as plsc

_NC = 2
_NS = 16
_NW = _NC * _NS
_LANES = 16

_R = 16
_NBUF = 4


def _tc_part_full(x, W, l_tc):
    b, l, d = x.shape
    BLK_L = 2048

    def body(x_ref, w_ref, o_ref):
        o_ref[...] = x_ref[...] + w_ref[...]

    return pl.pallas_call(
        body,
        grid=(l_tc // BLK_L, b),
        in_specs=[
            pl.BlockSpec((1, BLK_L, d), lambda i, j: (j, i, 0)),
            pl.BlockSpec((BLK_L, d), lambda i, j: (i, 0)),
        ],
        out_specs=pl.BlockSpec((1, BLK_L, d), lambda i, j: (j, i, 0)),
        out_shape=jax.ShapeDtypeStruct((b, l, d), x.dtype),
    )(x, W)


def _sc_part(x, W, l_tc):
    b, l, d = x.shape
    l_sc = l - l_tc
    l_per_w = l_sc // _NW
    n_chunks = l_per_w // _R
    units = n_chunks * b
    jpr = d // _LANES
    assert b == _NBUF and n_chunks % 2 == 0

    mesh = plsc.VectorSubcoreMesh(core_axis_name="c", subcore_axis_name="s")

    @functools.partial(
        pl.kernel,
        mesh=mesh,
        out_type=jax.ShapeDtypeStruct((b, l_sc, d), jnp.float32),
        scratch_types=[
            pltpu.VMEM((_NBUF, _R, d), jnp.float32),
            pltpu.VMEM((2, _R, d), jnp.float32),
        ] + [pltpu.SemaphoreType.DMA] * (2 * _NBUF + 2),
    )
    def sc_add(x_hbm, w_hbm, o_hbm, x_ring, w_ring, *sems):
        in_sems = sems[:_NBUF]
        out_sems = sems[_NBUF:2 * _NBUF]
        w_sems = sems[2 * _NBUF:]

        wid = lax.axis_index("s") * _NC + lax.axis_index("c")
        o_base = wid * l_per_w
        l_base = l_tc + o_base

        def start_in(u, slot):
            c = u // b
            bb = u % b
            lo = l_base + c * _R
            pltpu.async_copy(x_hbm.at[bb, pl.ds(lo, _R)], x_ring.at[slot],
                             in_sems[slot])

        def wait_in(slot):
            pltpu.make_async_copy(x_hbm.at[0, pl.ds(0, _R)],
                                  x_ring.at[slot], in_sems[slot]).wait()

        def start_out(u, slot):
            c = u // b
            bb = u % b
            oo = o_base + c * _R
            pltpu.async_copy(x_ring.at[slot], o_hbm.at[bb, pl.ds(oo, _R)],
                             out_sems[slot])

        def wait_out(slot):
            pltpu.make_async_copy(x_ring.at[slot],
                                  o_hbm.at[0, pl.ds(0, _R)],
                                  out_sems[slot]).wait()

        def start_w(c, slot):
            pltpu.async_copy(w_hbm.at[pl.ds(l_base + c * _R, _R)],
                             w_ring.at[slot], w_sems[slot])

        def wait_w(slot):
            pltpu.make_async_copy(w_hbm.at[pl.ds(0, _R)],
                                  w_ring.at[slot], w_sems[slot]).wait()

        start_w(0, 0)
        start_w(1, 1)
        for u0 in range(_NBUF - 1):
            start_in(u0, u0)

        @pl.loop(0, n_chunks, step=2)
        def block_body(c_base):
            for j in range(2 * b):
                u = c_base * b + j
                c = c_base + j // b
                bb = j % b
                wslot = (j // b) % 2
                xslot = j % _NBUF
                if bb == 0:
                    wait_w(wslot)

                wait_in(xslot)

                @plsc.parallel_loop(0, _R, 1, unroll=2)
                def add_body(r):
                    for jj in range(jpr):
                        sl = pl.ds(jj * _LANES, _LANES)
                        plsc.addupdate(x_ring.at[xslot, r, sl],
                                       w_ring[wslot, r, sl])

                if bb == b - 1:
                    @pl.when(c + 2 < n_chunks)
                    def _():
                        start_w(c + 2, wslot)

                start_out(u, xslot)

                v = u + _NBUF - 1
                vslot = (j + _NBUF - 1) % _NBUF

                @pl.when(v < units)
                def _():
                    @pl.when(u > 0)
                    def _():
                        wait_out(vslot)

                    start_in(v, vslot)

        for u in range(units - _NBUF, units):
            wait_out(u % _NBUF)

    return sc_add(x, W)


def kernel(x, W):
    l_tc = 6144
    tc_full = _tc_part_full(x, W, l_tc)
    sc = _sc_part(x, W, l_tc)
    return lax.dynamic_update_slice(tc_full, sc, (0, l_tc, 0))


# speedup vs baseline: 2.2648x; 1.4428x over previous
import jax
import jax.numpy as jnp
from jax.experimental import pallas as pl


def kernel(x, W):
    b, l, d = x.shape
    BLK_L = 2048

    def body(x_ref, w_ref, o_ref):
        o_ref[...] = x_ref[...] + w_ref[...]

    return pl.pallas_call(
        body,
        grid=(l // BLK_L, b),
        in_specs=[
            pl.BlockSpec((1, BLK_L, d), lambda i, j: (j, i, 0)),
            pl.BlockSpec((BLK_L, d), lambda i, j: (i, 0)),
        ],
        out_specs=pl.BlockSpec((1, BLK_L, d), lambda i, j: (j, i, 0)),
        out_shape=jax.ShapeDtypeStruct(x.shape, x.dtype),
    )(x, W)
